# Initial kernel scaffold; baseline (speedup 1.0000x reference)
import functools

import jax
import jax.numpy as jnp
from jax import lax
from jax.experimental import pallas as pl
from jax.experimental.pallas import tpu as pltpu
from jax.experimental.pallas import tpu_sc as plsc

N = 10000
D_IN = 128
D_OUT = 16
E = 320000

NC = 2
NS = 16
NW = NC * NS
LANES = 16
NPAD = 10240
RPS = NPAD // NS
EPW = E // NW
CH = 80
KCH = EPW // CH

_mesh = plsc.VectorSubcoreMesh(core_axis_name="c", subcore_axis_name="s")


@functools.partial(
    pl.kernel,
    out_type=jax.ShapeDtypeStruct((NC, NPAD), jnp.float32),
    mesh=_mesh,
    scratch_types=[
        pltpu.VMEM((EPW,), jnp.int32),
        pltpu.VMEM((NPAD,), jnp.float32),
        pltpu.VMEM((RPS,), jnp.float32),
        pltpu.VMEM((RPS,), jnp.float32),
        pltpu.VMEM_SHARED((NS, NPAD), jnp.float32),
    ],
)
def _deg_kernel(dst_hbm, out_hbm, idx_v, deg_v, comb_v, tmp_v, pub_sh):
    cid = lax.axis_index("c")
    sid = lax.axis_index("s")
    wid = cid * NS + sid

    pltpu.sync_copy(dst_hbm.at[wid], idx_v)

    def _zero(i, carry):
        deg_v[pl.ds(i * LANES, LANES)] = jnp.zeros((LANES,), jnp.float32)
        return carry
    lax.fori_loop(0, NPAD // LANES, _zero, 0)

    ones = jnp.ones((LANES,), jnp.float32)

    def _count(i, carry):
        idx = idx_v[pl.ds(i * LANES, LANES)]
        plsc.addupdate_scatter(deg_v, [idx], ones)
        return carry
    lax.fori_loop(0, EPW // LANES, _count, 0)

    pltpu.sync_copy(deg_v, pub_sh.at[sid])
    plsc.subcore_barrier()

    base = sid * RPS
    pltpu.sync_copy(pub_sh.at[0, pl.ds(base, RPS)], comb_v)
    for t in range(1, NS):
        pltpu.sync_copy(pub_sh.at[t, pl.ds(base, RPS)], tmp_v)

        def _acc(j, carry):
            sl = pl.ds(j * LANES, LANES)
            comb_v[sl] = comb_v[sl] + tmp_v[sl]
            return carry
        lax.fori_loop(0, RPS // LANES, _acc, 0)

    pltpu.sync_copy(comb_v, out_hbm.at[cid, pl.ds(base, RPS)])


def _mm_body(x_ref, w_ref, degp_ref, g_ref):
    h = jnp.dot(x_ref[...], w_ref[...], preferred_element_type=jnp.float32)
    deg = degp_ref[0, :] + degp_ref[1, :] + 1.0
    g_ref[...] = h * lax.rsqrt(deg)[:, None]


def _mm_call(x, w, degp):
    blk = 1000
    grid = N // blk
    return pl.pallas_call(
        _mm_body,
        grid=(grid,),
        in_specs=[
            pl.BlockSpec((blk, D_IN), lambda i: (i, 0)),
            pl.BlockSpec((D_IN, D_OUT), lambda i: (0, 0)),
            pl.BlockSpec((NC, blk), lambda i: (0, i)),
        ],
        out_specs=pl.BlockSpec((blk, D_OUT), lambda i: (i, 0)),
        out_shape=jax.ShapeDtypeStruct((N, D_OUT), jnp.float32),
    )(x, w, degp)


@functools.partial(
    pl.kernel,
    out_type=jax.ShapeDtypeStruct((NC, NPAD, D_OUT), jnp.float32),
    mesh=_mesh,
    scratch_types=[
        pltpu.VMEM((KCH, CH), jnp.int32),
        pltpu.VMEM((KCH, CH), jnp.int32),
        pltpu.VMEM((CH, D_OUT), jnp.float32),
        pltpu.VMEM((RPS, D_OUT), jnp.float32),
        pltpu.VMEM_SHARED((NPAD, D_OUT), jnp.float32),
        pltpu.SemaphoreType.DMA,
    ],
)
def _agg_kernel(src_hbm, dst_hbm, g_hbm, out_hbm,
                src_v, dst_v, rows_v, buf_v, acc_sh, sem):
    cid = lax.axis_index("c")
    sid = lax.axis_index("s")
    wid = cid * NS + sid

    pltpu.sync_copy(src_hbm.at[wid], src_v)
    pltpu.sync_copy(dst_hbm.at[wid], dst_v)

    def _zero(i, carry):
        buf_v[i] = jnp.zeros((D_OUT,), jnp.float32)
        return carry
    lax.fori_loop(0, RPS, _zero, 0)

    base = sid * RPS
    pltpu.sync_copy(buf_v, acc_sh.at[pl.ds(base, RPS)])
    plsc.subcore_barrier()

    def _chunk(k, carry):
        pltpu.async_copy(g_hbm.at[src_v.at[k]], rows_v, sem).wait()
        pltpu.sync_copy(rows_v, acc_sh.at[dst_v.at[k]], add=True)
        return carry
    lax.fori_loop(0, KCH, _chunk, 0)

    plsc.subcore_barrier()
    pltpu.sync_copy(acc_sh.at[pl.ds(base, RPS)], buf_v)
    pltpu.sync_copy(buf_v, out_hbm.at[cid, pl.ds(base, RPS), :])


def _fin_body(accp_ref, g_ref, degp_ref, b_ref, out_ref):
    deg = degp_ref[0, :] + degp_ref[1, :] + 1.0
    s = accp_ref[0, :, :] + accp_ref[1, :, :] + g_ref[...]
    out_ref[...] = s * lax.rsqrt(deg)[:, None] + b_ref[0, :]


def _fin_call(accp, g, degp, b):
    blk = 1000
    grid = N // blk
    return pl.pallas_call(
        _fin_body,
        grid=(grid,),
        in_specs=[
            pl.BlockSpec((NC, blk, D_OUT), lambda i: (0, i, 0)),
            pl.BlockSpec((blk, D_OUT), lambda i: (i, 0)),
            pl.BlockSpec((NC, blk), lambda i: (0, i)),
            pl.BlockSpec((1, D_OUT), lambda i: (0, 0)),
        ],
        out_specs=pl.BlockSpec((blk, D_OUT), lambda i: (i, 0)),
        out_shape=jax.ShapeDtypeStruct((N, D_OUT), jnp.float32),
    )(accp, g, degp, b)


def kernel(x, edge_index, W, b):
    src = edge_index[0].reshape(NW, KCH, CH)
    dst = edge_index[1].reshape(NW, KCH, CH)
    dst_flat = edge_index[1].reshape(NW, EPW)

    degp = _deg_kernel(dst_flat)
    g = _mm_call(x, W, degp)
    accp = _agg_kernel(src, dst, g)
    out = _fin_call(accp, g, degp, b.reshape(1, D_OUT))
    return out


# speedup vs baseline: 25.0314x; 25.0314x over previous
import functools

import jax
import jax.numpy as jnp
from jax import lax
from jax.experimental import pallas as pl
from jax.experimental.pallas import tpu as pltpu
from jax.experimental.pallas import tpu_sc as plsc

N = 10000
D_IN = 128
D_OUT = 16
E = 320000

NC = 2
NS = 16
NW = NC * NS
LANES = 16
NPAD = 10240
RPS = NPAD // NS
EPW = E // NW
CH = 80
KCH = EPW // CH

_mesh = plsc.VectorSubcoreMesh(
    core_axis_name="c", subcore_axis_name="s", num_cores=NC, num_subcores=NS
)
_CP = pltpu.CompilerParams(needs_layout_passes=False)
_CP2 = pltpu.CompilerParams(needs_layout_passes=False, use_tc_tiling_on_sc=False)


@functools.partial(
    pl.kernel,
    out_type=jax.ShapeDtypeStruct((NC, NPAD), jnp.float32),
    mesh=_mesh,
    compiler_params=_CP,
    scratch_types=[
        pltpu.VMEM((EPW,), jnp.int32),
        pltpu.VMEM((NPAD,), jnp.float32),
        pltpu.VMEM((RPS,), jnp.float32),
        pltpu.VMEM((RPS,), jnp.float32),
        pltpu.VMEM_SHARED((NS, NPAD), jnp.float32),
    ],
)
def _deg_kernel(dst_hbm, out_hbm, idx_v, deg_v, comb_v, tmp_v, pub_sh):
    cid = lax.axis_index("c")
    sid = lax.axis_index("s")
    wid = cid * NS + sid

    pltpu.sync_copy(dst_hbm.at[pl.ds(wid * EPW, EPW)], idx_v)

    def _zero(i, carry):
        deg_v[pl.ds(i * LANES, LANES)] = jnp.zeros((LANES,), jnp.float32)
        return carry
    lax.fori_loop(0, NPAD // LANES, _zero, 0)

    ones = jnp.ones((LANES,), jnp.float32)

    def _count(i, carry):
        idx = idx_v[pl.ds(i * LANES, LANES)]
        plsc.addupdate_scatter(deg_v, [idx], ones)
        return carry
    lax.fori_loop(0, EPW // LANES, _count, 0)

    pltpu.sync_copy(deg_v, pub_sh.at[sid])
    plsc.subcore_barrier()

    base = sid * RPS
    pltpu.sync_copy(pub_sh.at[0, pl.ds(base, RPS)], comb_v)
    for t in range(1, NS):
        pltpu.sync_copy(pub_sh.at[t, pl.ds(base, RPS)], tmp_v)

        def _acc(j, carry):
            sl = pl.ds(j * LANES, LANES)
            comb_v[sl] = comb_v[sl] + tmp_v[sl]
            return carry
        lax.fori_loop(0, RPS // LANES, _acc, 0)

    pltpu.sync_copy(comb_v, out_hbm.at[cid, pl.ds(base, RPS)])


def _mm_body(x_ref, w_ref, degp_ref, g_ref):
    h = jnp.dot(x_ref[...], w_ref[...], preferred_element_type=jnp.float32)
    deg = degp_ref[0, :, :] + degp_ref[1, :, :] + 1.0
    g_ref[...] = h * lax.rsqrt(deg)


def _mm_call(x, w, degp):
    blk = 1024
    grid = NPAD // blk
    return pl.pallas_call(
        _mm_body,
        grid=(grid,),
        in_specs=[
            pl.BlockSpec((blk, D_IN), lambda i: (i, 0)),
            pl.BlockSpec((D_IN, D_OUT), lambda i: (0, 0)),
            pl.BlockSpec((NC, blk, 1), lambda i: (0, i, 0)),
        ],
        out_specs=pl.BlockSpec((blk, D_OUT), lambda i: (i, 0)),
        out_shape=jax.ShapeDtypeStruct((NPAD, D_OUT), jnp.float32),
    )(x, w, degp)


@functools.partial(
    pl.kernel,
    out_type=jax.ShapeDtypeStruct((NC, NPAD, D_OUT), jnp.float32),
    mesh=_mesh,
    compiler_params=_CP2,
    scratch_types=[
        pltpu.VMEM((1, CH), jnp.int32),
        pltpu.VMEM((1, CH), jnp.int32),
        pltpu.VMEM((CH, D_OUT), jnp.float32),
        pltpu.VMEM((RPS, D_OUT), jnp.float32),
        pltpu.VMEM_SHARED((NPAD, D_OUT), jnp.float32),
        pltpu.SemaphoreType.DMA,
    ],
)
def _agg_kernel(src_hbm, dst_hbm, g_hbm, out_hbm,
                src_v, dst_v, rows_v, buf_v, acc_sh, sem):
    cid = lax.axis_index("c")
    sid = lax.axis_index("s")
    wid = cid * NS + sid
    base = sid * RPS
    ebase = wid * EPW

    def _zero(i, carry):
        buf_v[i] = jnp.zeros((D_OUT,), jnp.float32)
        return carry
    lax.fori_loop(0, RPS, _zero, 0)
    pltpu.sync_copy(buf_v, acc_sh.at[pl.ds(base, RPS)])
    plsc.subcore_barrier()

    def _chunk(k, carry):
        off = ebase + k * CH
        pltpu.sync_copy(src_hbm.at[pl.ds(off, CH)], src_v.at[0])
        pltpu.sync_copy(dst_hbm.at[pl.ds(off, CH)], dst_v.at[0])
        pltpu.async_copy(g_hbm.at[src_v.at[0]], rows_v, sem).wait()
        pltpu.sync_copy(rows_v, acc_sh.at[dst_v.at[0]], add=True)
        return carry
    lax.fori_loop(0, KCH, _chunk, 0)

    plsc.subcore_barrier()
    pltpu.sync_copy(acc_sh.at[pl.ds(base, RPS)], buf_v)
    pltpu.sync_copy(buf_v, out_hbm.at[cid, pl.ds(base, RPS), :])


def _fin_body(accp_ref, g_ref, degp_ref, b_ref, out_ref):
    deg = degp_ref[0, :, :] + degp_ref[1, :, :] + 1.0
    s = accp_ref[0, :, :] + accp_ref[1, :, :] + g_ref[...]
    out_ref[...] = s * lax.rsqrt(deg) + b_ref[0, :]


def _fin_call(accp, g, degp, b):
    blk = 1024
    grid = NPAD // blk
    return pl.pallas_call(
        _fin_body,
        grid=(grid,),
        in_specs=[
            pl.BlockSpec((NC, blk, D_OUT), lambda i: (0, i, 0)),
            pl.BlockSpec((blk, D_OUT), lambda i: (i, 0)),
            pl.BlockSpec((NC, blk, 1), lambda i: (0, i, 0)),
            pl.BlockSpec((1, D_OUT), lambda i: (0, 0)),
        ],
        out_specs=pl.BlockSpec((blk, D_OUT), lambda i: (i, 0)),
        out_shape=jax.ShapeDtypeStruct((N, D_OUT), jnp.float32),
    )(accp, g, degp, b)


def kernel(x, edge_index, W, b):
    src = edge_index[0]
    dst = edge_index[1]

    degp = _deg_kernel(dst).reshape(NC, NPAD, 1)
    g = _mm_call(x, W, degp)
    accp = _agg_kernel(src, dst, g)
    out = _fin_call(accp, g, degp, b.reshape(1, D_OUT))
    return out


# speedup vs baseline: 62.6106x; 2.5013x over previous
import functools

import jax
import jax.numpy as jnp
from jax import lax
from jax.experimental import pallas as pl
from jax.experimental.pallas import tpu as pltpu
from jax.experimental.pallas import tpu_sc as plsc

N = 10000
D_IN = 128
D_OUT = 16
E = 320000

NC = 2
NS = 16
NW = NC * NS
LANES = 16
NPAD = 10240
RPS = NPAD // NS
EPW = E // NW
CH = 400
KCH = EPW // CH
NBUF = 2

_mesh = plsc.VectorSubcoreMesh(
    core_axis_name="c", subcore_axis_name="s", num_cores=NC, num_subcores=NS
)
_CP = pltpu.CompilerParams(needs_layout_passes=False)
_CP2 = pltpu.CompilerParams(needs_layout_passes=False, use_tc_tiling_on_sc=False)


@functools.partial(
    pl.kernel,
    out_type=jax.ShapeDtypeStruct((NC, NPAD), jnp.float32),
    mesh=_mesh,
    compiler_params=_CP,
    scratch_types=[
        pltpu.VMEM((EPW,), jnp.int32),
        pltpu.VMEM((NPAD,), jnp.float32),
        pltpu.VMEM((RPS,), jnp.float32),
        pltpu.VMEM((RPS,), jnp.float32),
        pltpu.VMEM_SHARED((NS, NPAD), jnp.float32),
    ],
)
def _deg_kernel(dst_hbm, out_hbm, idx_v, deg_v, comb_v, tmp_v, pub_sh):
    cid = lax.axis_index("c")
    sid = lax.axis_index("s")
    wid = cid * NS + sid

    pltpu.sync_copy(dst_hbm.at[pl.ds(wid * EPW, EPW)], idx_v)

    def _zero(i, carry):
        deg_v[pl.ds(i * LANES, LANES)] = jnp.zeros((LANES,), jnp.float32)
        return carry
    lax.fori_loop(0, NPAD // LANES, _zero, 0)

    ones = jnp.ones((LANES,), jnp.float32)

    def _count(i, carry):
        idx = idx_v[pl.ds(i * LANES, LANES)]
        plsc.addupdate_scatter(deg_v, [idx], ones)
        return carry
    lax.fori_loop(0, EPW // LANES, _count, 0)

    pltpu.sync_copy(deg_v, pub_sh.at[sid])
    plsc.subcore_barrier()

    base = sid * RPS
    pltpu.sync_copy(pub_sh.at[0, pl.ds(base, RPS)], comb_v)
    for t in range(1, NS):
        pltpu.sync_copy(pub_sh.at[t, pl.ds(base, RPS)], tmp_v)

        def _acc(j, carry):
            sl = pl.ds(j * LANES, LANES)
            comb_v[sl] = comb_v[sl] + tmp_v[sl]
            return carry
        lax.fori_loop(0, RPS // LANES, _acc, 0)

    pltpu.sync_copy(comb_v, out_hbm.at[cid, pl.ds(base, RPS)])


def _mm_body(x_ref, w_ref, degp_ref, g_ref):
    h = jnp.dot(x_ref[...], w_ref[...], preferred_element_type=jnp.float32)
    deg = degp_ref[0, :, :] + degp_ref[1, :, :] + 1.0
    g_ref[...] = h * lax.rsqrt(deg)


def _mm_call(x, w, degp):
    blk = 1024
    grid = NPAD // blk
    return pl.pallas_call(
        _mm_body,
        grid=(grid,),
        in_specs=[
            pl.BlockSpec((blk, D_IN), lambda i: (i, 0)),
            pl.BlockSpec((D_IN, D_OUT), lambda i: (0, 0)),
            pl.BlockSpec((NC, blk, 1), lambda i: (0, i, 0)),
        ],
        out_specs=pl.BlockSpec((blk, D_OUT), lambda i: (i, 0)),
        out_shape=jax.ShapeDtypeStruct((NPAD, D_OUT), jnp.float32),
    )(x, w, degp)


@functools.partial(
    pl.kernel,
    out_type=jax.ShapeDtypeStruct((NC, NPAD, D_OUT), jnp.float32),
    mesh=_mesh,
    compiler_params=_CP2,
    scratch_types=[
        pltpu.VMEM((KCH, CH), jnp.int32),
        pltpu.VMEM((KCH, CH), jnp.int32),
        pltpu.VMEM((NBUF, CH, D_OUT), jnp.float32),
        pltpu.VMEM((RPS, D_OUT), jnp.float32),
        pltpu.VMEM_SHARED((NPAD, D_OUT), jnp.float32),
        pltpu.SemaphoreType.DMA,
        pltpu.SemaphoreType.DMA,
    ],
)
def _agg_kernel(src_hbm, dst_hbm, g_hbm, out_hbm,
                src_v, dst_v, rows_v, buf_v, acc_sh, sem0, sem1):
    cid = lax.axis_index("c")
    sid = lax.axis_index("s")
    wid = cid * NS + sid
    base = sid * RPS
    sems = (sem0, sem1)

    pltpu.sync_copy(src_hbm.at[wid], src_v)
    pltpu.sync_copy(dst_hbm.at[wid], dst_v)

    def _zero(i, carry):
        buf_v[i] = jnp.zeros((D_OUT,), jnp.float32)
        return carry
    lax.fori_loop(0, RPS, _zero, 0)
    pltpu.sync_copy(buf_v, acc_sh.at[pl.ds(base, RPS)])
    plsc.subcore_barrier()

    pltpu.async_copy(g_hbm.at[src_v.at[0]], rows_v.at[0], sem0)

    def _chunk(k, carry):
        p = lax.rem(k, NBUF)

        @pl.when(k + 1 < KCH)
        def _():
            pn = lax.rem(k + 1, NBUF)
            for q in range(NBUF):
                @pl.when(pn == q)
                def _():
                    pltpu.async_copy(
                        g_hbm.at[src_v.at[k + 1]], rows_v.at[q], sems[q])

        for q in range(NBUF):
            @pl.when(p == q)
            def _():
                pltpu.make_async_copy(
                    g_hbm.at[src_v.at[k]], rows_v.at[q], sems[q]).wait()
                pltpu.sync_copy(rows_v.at[q], acc_sh.at[dst_v.at[k]], add=True)
        return carry
    lax.fori_loop(0, KCH, _chunk, 0)

    plsc.subcore_barrier()
    pltpu.sync_copy(acc_sh.at[pl.ds(base, RPS)], buf_v)
    pltpu.sync_copy(buf_v, out_hbm.at[cid, pl.ds(base, RPS), :])


def _fin_body(accp_ref, g_ref, degp_ref, b_ref, out_ref):
    deg = degp_ref[0, :, :] + degp_ref[1, :, :] + 1.0
    s = accp_ref[0, :, :] + accp_ref[1, :, :] + g_ref[...]
    out_ref[...] = s * lax.rsqrt(deg) + b_ref[0, :]


def _fin_call(accp, g, degp, b):
    blk = 1024
    grid = NPAD // blk
    return pl.pallas_call(
        _fin_body,
        grid=(grid,),
        in_specs=[
            pl.BlockSpec((NC, blk, D_OUT), lambda i: (0, i, 0)),
            pl.BlockSpec((blk, D_OUT), lambda i: (i, 0)),
            pl.BlockSpec((NC, blk, 1), lambda i: (0, i, 0)),
            pl.BlockSpec((1, D_OUT), lambda i: (0, 0)),
        ],
        out_specs=pl.BlockSpec((blk, D_OUT), lambda i: (i, 0)),
        out_shape=jax.ShapeDtypeStruct((N, D_OUT), jnp.float32),
    )(accp, g, degp, b)


def kernel(x, edge_index, W, b):
    src = edge_index[0]
    dst = edge_index[1]
    src3 = src.reshape(NW, KCH, CH)
    dst3 = dst.reshape(NW, KCH, CH)

    degp = _deg_kernel(dst).reshape(NC, NPAD, 1)
    g = _mm_call(x, W, degp)
    accp = _agg_kernel(src3, dst3, g)
    out = _fin_call(accp, g, degp, b.reshape(1, D_OUT))
    return out


# speedup vs baseline: 65.0843x; 1.0395x over previous
import functools

import jax
import jax.numpy as jnp
from jax import lax
from jax.experimental import pallas as pl
from jax.experimental.pallas import tpu as pltpu
from jax.experimental.pallas import tpu_sc as plsc

N = 10000
D_IN = 128
D_OUT = 16
E = 320000

NC = 2
NS = 16
NW = NC * NS
LANES = 16
NPAD = 10240
RPS = NPAD // NS
EPW = E // NW
CH = 1000
KCH = EPW // CH
NBUF = 2

_mesh = plsc.VectorSubcoreMesh(
    core_axis_name="c", subcore_axis_name="s", num_cores=NC, num_subcores=NS
)
_CP = pltpu.CompilerParams(needs_layout_passes=False)
_CP2 = pltpu.CompilerParams(needs_layout_passes=False, use_tc_tiling_on_sc=False)


@functools.partial(
    pl.kernel,
    out_type=jax.ShapeDtypeStruct((NC, NPAD), jnp.float32),
    mesh=_mesh,
    compiler_params=_CP,
    scratch_types=[
        pltpu.VMEM((EPW,), jnp.int32),
        pltpu.VMEM((NPAD,), jnp.float32),
        pltpu.VMEM((RPS,), jnp.float32),
        pltpu.VMEM((RPS,), jnp.float32),
        pltpu.VMEM_SHARED((NS, NPAD), jnp.float32),
    ],
)
def _deg_kernel(dst_hbm, out_hbm, idx_v, deg_v, comb_v, tmp_v, pub_sh):
    cid = lax.axis_index("c")
    sid = lax.axis_index("s")
    wid = cid * NS + sid

    pltpu.sync_copy(dst_hbm.at[pl.ds(wid * EPW, EPW)], idx_v)

    def _zero(i, carry):
        deg_v[pl.ds(i * LANES, LANES)] = jnp.zeros((LANES,), jnp.float32)
        return carry
    lax.fori_loop(0, NPAD // LANES, _zero, 0)

    ones = jnp.ones((LANES,), jnp.float32)

    def _count(i, carry):
        idx = idx_v[pl.ds(i * LANES, LANES)]
        plsc.addupdate_scatter(deg_v, [idx], ones)
        return carry
    lax.fori_loop(0, EPW // LANES, _count, 0, unroll=8)

    pltpu.sync_copy(deg_v, pub_sh.at[sid])
    plsc.subcore_barrier()

    base = sid * RPS
    pltpu.sync_copy(pub_sh.at[0, pl.ds(base, RPS)], comb_v)
    for t in range(1, NS):
        pltpu.sync_copy(pub_sh.at[t, pl.ds(base, RPS)], tmp_v)

        def _acc(j, carry):
            sl = pl.ds(j * LANES, LANES)
            comb_v[sl] = comb_v[sl] + tmp_v[sl]
            return carry
        lax.fori_loop(0, RPS // LANES, _acc, 0)

    pltpu.sync_copy(comb_v, out_hbm.at[cid, pl.ds(base, RPS)])


def _mm_body(x_ref, w_ref, degp_ref, g_ref):
    h = jnp.dot(x_ref[...], w_ref[...], preferred_element_type=jnp.float32)
    deg = degp_ref[0, :, :] + degp_ref[1, :, :] + 1.0
    g_ref[...] = h * lax.rsqrt(deg)


def _mm_call(x, w, degp):
    blk = 1024
    grid = NPAD // blk
    return pl.pallas_call(
        _mm_body,
        grid=(grid,),
        in_specs=[
            pl.BlockSpec((blk, D_IN), lambda i: (i, 0)),
            pl.BlockSpec((D_IN, D_OUT), lambda i: (0, 0)),
            pl.BlockSpec((NC, blk, 1), lambda i: (0, i, 0)),
        ],
        out_specs=pl.BlockSpec((blk, D_OUT), lambda i: (i, 0)),
        out_shape=jax.ShapeDtypeStruct((NPAD, D_OUT), jnp.float32),
    )(x, w, degp)


@functools.partial(
    pl.kernel,
    out_type=jax.ShapeDtypeStruct((NC, NPAD, D_OUT), jnp.float32),
    mesh=_mesh,
    compiler_params=_CP2,
    scratch_types=[
        pltpu.VMEM((KCH, CH), jnp.int32),
        pltpu.VMEM((KCH, CH), jnp.int32),
        pltpu.VMEM((NBUF, CH, D_OUT), jnp.float32),
        pltpu.VMEM((RPS, D_OUT), jnp.float32),
        pltpu.VMEM_SHARED((NPAD, D_OUT), jnp.float32),
        pltpu.SemaphoreType.DMA,
        pltpu.SemaphoreType.DMA,
    ],
)
def _agg_kernel(src_hbm, dst_hbm, g_hbm, out_hbm,
                src_v, dst_v, rows_v, buf_v, acc_sh, sem0, sem1):
    cid = lax.axis_index("c")
    sid = lax.axis_index("s")
    wid = cid * NS + sid
    base = sid * RPS
    sems = (sem0, sem1)

    pltpu.sync_copy(src_hbm.at[wid], src_v)
    pltpu.sync_copy(dst_hbm.at[wid], dst_v)

    def _zero(i, carry):
        buf_v[i] = jnp.zeros((D_OUT,), jnp.float32)
        return carry
    lax.fori_loop(0, RPS, _zero, 0)
    pltpu.sync_copy(buf_v, acc_sh.at[pl.ds(base, RPS)])
    plsc.subcore_barrier()

    pltpu.async_copy(g_hbm.at[src_v.at[0]], rows_v.at[0], sem0)

    def _chunk(k, carry):
        p = lax.rem(k, NBUF)

        @pl.when(k + 1 < KCH)
        def _():
            pn = lax.rem(k + 1, NBUF)
            for q in range(NBUF):
                @pl.when(pn == q)
                def _():
                    pltpu.async_copy(
                        g_hbm.at[src_v.at[k + 1]], rows_v.at[q], sems[q])

        for q in range(NBUF):
            @pl.when(p == q)
            def _():
                pltpu.make_async_copy(
                    g_hbm.at[src_v.at[k]], rows_v.at[q], sems[q]).wait()
                pltpu.sync_copy(rows_v.at[q], acc_sh.at[dst_v.at[k]], add=True)
        return carry
    lax.fori_loop(0, KCH, _chunk, 0)

    plsc.subcore_barrier()
    pltpu.sync_copy(acc_sh.at[pl.ds(base, RPS)], buf_v)
    pltpu.sync_copy(buf_v, out_hbm.at[cid, pl.ds(base, RPS), :])


def _fin_body(accp_ref, g_ref, degp_ref, b_ref, out_ref):
    deg = degp_ref[0, :, :] + degp_ref[1, :, :] + 1.0
    s = accp_ref[0, :, :] + accp_ref[1, :, :] + g_ref[...]
    out_ref[...] = s * lax.rsqrt(deg) + b_ref[0, :]


def _fin_call(accp, g, degp, b):
    blk = 1024
    grid = NPAD // blk
    return pl.pallas_call(
        _fin_body,
        grid=(grid,),
        in_specs=[
            pl.BlockSpec((NC, blk, D_OUT), lambda i: (0, i, 0)),
            pl.BlockSpec((blk, D_OUT), lambda i: (i, 0)),
            pl.BlockSpec((NC, blk, 1), lambda i: (0, i, 0)),
            pl.BlockSpec((1, D_OUT), lambda i: (0, 0)),
        ],
        out_specs=pl.BlockSpec((blk, D_OUT), lambda i: (i, 0)),
        out_shape=jax.ShapeDtypeStruct((N, D_OUT), jnp.float32),
    )(accp, g, degp, b)


def kernel(x, edge_index, W, b):
    src = edge_index[0]
    dst = edge_index[1]
    src3 = src.reshape(NW, KCH, CH)
    dst3 = dst.reshape(NW, KCH, CH)

    degp = _deg_kernel(dst).reshape(NC, NPAD, 1)
    g = _mm_call(x, W, degp)
    accp = _agg_kernel(src3, dst3, g)
    out = _fin_call(accp, g, degp, b.reshape(1, D_OUT))
    return out


# speedup vs baseline: 66.7208x; 1.0251x over previous
import functools

import jax
import jax.numpy as jnp
from jax import lax
from jax.experimental import pallas as pl
from jax.experimental.pallas import tpu as pltpu
from jax.experimental.pallas import tpu_sc as plsc

N = 10000
D_IN = 128
D_OUT = 16
E = 320000

NC = 2
NS = 16
NW = NC * NS
LANES = 16
NPAD = 10240
RPS = NPAD // NS
EPW = E // NW
CH = 2000
KCH = EPW // CH
NBUF = 2

_mesh = plsc.VectorSubcoreMesh(
    core_axis_name="c", subcore_axis_name="s", num_cores=NC, num_subcores=NS
)
_CP = pltpu.CompilerParams(needs_layout_passes=False)
_CP2 = pltpu.CompilerParams(needs_layout_passes=False, use_tc_tiling_on_sc=False)


@functools.partial(
    pl.kernel,
    out_type=jax.ShapeDtypeStruct((NC, NPAD), jnp.float32),
    mesh=_mesh,
    compiler_params=_CP,
    scratch_types=[
        pltpu.VMEM((EPW,), jnp.int32),
        pltpu.VMEM((NPAD,), jnp.float32),
        pltpu.VMEM((RPS,), jnp.float32),
        pltpu.VMEM((NS, RPS), jnp.float32),
        pltpu.VMEM_SHARED((NS, NPAD), jnp.float32),
    ],
)
def _deg_kernel(dst_hbm, out_hbm, idx_v, deg_v, comb_v, tmp_v, pub_sh):
    cid = lax.axis_index("c")
    sid = lax.axis_index("s")
    wid = cid * NS + sid

    pltpu.sync_copy(dst_hbm.at[pl.ds(wid * EPW, EPW)], idx_v)

    def _zero(i, carry):
        deg_v[pl.ds(i * LANES, LANES)] = jnp.zeros((LANES,), jnp.float32)
        return carry
    lax.fori_loop(0, NPAD // LANES, _zero, 0)

    ones = jnp.ones((LANES,), jnp.float32)

    def _count(i, carry):
        idx = idx_v[pl.ds(i * LANES, LANES)]
        plsc.addupdate_scatter(deg_v, [idx], ones)
        return carry
    lax.fori_loop(0, EPW // LANES, _count, 0, unroll=8)

    pltpu.sync_copy(deg_v, pub_sh.at[sid])
    plsc.subcore_barrier()

    base = sid * RPS
    pltpu.sync_copy(pub_sh.at[:, pl.ds(base, RPS)], tmp_v)

    def _acc(j, carry):
        sl = pl.ds(j * LANES, LANES)
        v = tmp_v[0, sl]
        for t in range(1, NS):
            v = v + tmp_v[t, sl]
        comb_v[sl] = v
        return carry
    lax.fori_loop(0, RPS // LANES, _acc, 0, unroll=4)

    pltpu.sync_copy(comb_v, out_hbm.at[cid, pl.ds(base, RPS)])


def _mm_body(x_ref, w_ref, degp_ref, g_ref):
    h = jnp.dot(x_ref[...], w_ref[...], preferred_element_type=jnp.float32)
    deg = degp_ref[0, :, :] + degp_ref[1, :, :] + 1.0
    g_ref[...] = h * lax.rsqrt(deg)


def _mm_call(x, w, degp):
    blk = 1024
    grid = NPAD // blk
    return pl.pallas_call(
        _mm_body,
        grid=(grid,),
        in_specs=[
            pl.BlockSpec((blk, D_IN), lambda i: (i, 0)),
            pl.BlockSpec((D_IN, D_OUT), lambda i: (0, 0)),
            pl.BlockSpec((NC, blk, 1), lambda i: (0, i, 0)),
        ],
        out_specs=pl.BlockSpec((blk, D_OUT), lambda i: (i, 0)),
        out_shape=jax.ShapeDtypeStruct((NPAD, D_OUT), jnp.float32),
    )(x, w, degp)


@functools.partial(
    pl.kernel,
    out_type=jax.ShapeDtypeStruct((NC, NPAD, D_OUT), jnp.float32),
    mesh=_mesh,
    compiler_params=_CP2,
    scratch_types=[
        pltpu.VMEM((KCH, CH), jnp.int32),
        pltpu.VMEM((KCH, CH), jnp.int32),
        pltpu.VMEM((NBUF, CH, D_OUT), jnp.float32),
        pltpu.VMEM((RPS, D_OUT), jnp.float32),
        pltpu.VMEM_SHARED((NPAD, D_OUT), jnp.float32),
        pltpu.SemaphoreType.DMA,
        pltpu.SemaphoreType.DMA,
    ],
)
def _agg_kernel(src_hbm, dst_hbm, g_hbm, out_hbm,
                src_v, dst_v, rows_v, buf_v, acc_sh, sem0, sem1):
    cid = lax.axis_index("c")
    sid = lax.axis_index("s")
    wid = cid * NS + sid
    base = sid * RPS
    sems = (sem0, sem1)

    pltpu.sync_copy(src_hbm.at[wid], src_v)
    pltpu.sync_copy(dst_hbm.at[wid], dst_v)

    def _zero(i, carry):
        buf_v[i] = jnp.zeros((D_OUT,), jnp.float32)
        return carry
    lax.fori_loop(0, RPS, _zero, 0)
    pltpu.sync_copy(buf_v, acc_sh.at[pl.ds(base, RPS)])
    plsc.subcore_barrier()

    pltpu.async_copy(g_hbm.at[src_v.at[0]], rows_v.at[0], sem0)

    def _chunk(k, carry):
        p = lax.rem(k, NBUF)

        @pl.when(k + 1 < KCH)
        def _():
            pn = lax.rem(k + 1, NBUF)
            for q in range(NBUF):
                @pl.when(pn == q)
                def _():
                    pltpu.async_copy(
                        g_hbm.at[src_v.at[k + 1]], rows_v.at[q], sems[q])

        for q in range(NBUF):
            @pl.when(p == q)
            def _():
                pltpu.make_async_copy(
                    g_hbm.at[src_v.at[k]], rows_v.at[q], sems[q]).wait()
                pltpu.sync_copy(rows_v.at[q], acc_sh.at[dst_v.at[k]], add=True)
        return carry
    lax.fori_loop(0, KCH, _chunk, 0)

    plsc.subcore_barrier()
    pltpu.sync_copy(acc_sh.at[pl.ds(base, RPS)], buf_v)
    pltpu.sync_copy(buf_v, out_hbm.at[cid, pl.ds(base, RPS), :])


def _fin_body(accp_ref, g_ref, degp_ref, b_ref, out_ref):
    deg = degp_ref[0, :, :] + degp_ref[1, :, :] + 1.0
    s = accp_ref[0, :, :] + accp_ref[1, :, :] + g_ref[...]
    out_ref[...] = s * lax.rsqrt(deg) + b_ref[0, :]


def _fin_call(accp, g, degp, b):
    blk = 1024
    grid = NPAD // blk
    return pl.pallas_call(
        _fin_body,
        grid=(grid,),
        in_specs=[
            pl.BlockSpec((NC, blk, D_OUT), lambda i: (0, i, 0)),
            pl.BlockSpec((blk, D_OUT), lambda i: (i, 0)),
            pl.BlockSpec((NC, blk, 1), lambda i: (0, i, 0)),
            pl.BlockSpec((1, D_OUT), lambda i: (0, 0)),
        ],
        out_specs=pl.BlockSpec((blk, D_OUT), lambda i: (i, 0)),
        out_shape=jax.ShapeDtypeStruct((N, D_OUT), jnp.float32),
    )(accp, g, degp, b)


def kernel(x, edge_index, W, b):
    src = edge_index[0]
    dst = edge_index[1]
    src3 = src.reshape(NW, KCH, CH)
    dst3 = dst.reshape(NW, KCH, CH)

    degp = _deg_kernel(dst).reshape(NC, NPAD, 1)
    g = _mm_call(x, W, degp)
    accp = _agg_kernel(src3, dst3, g)
    out = _fin_call(accp, g, degp, b.reshape(1, D_OUT))
    return out
